# Initial kernel scaffold; baseline (speedup 1.0000x reference)
import jax
import jax.numpy as jnp
from jax.experimental import pallas as pl


def kernel(x, edge_index, edge_weight, W_l, W_r, bias):
    raise NotImplementedError("write your pallas kernel here")


# speedup vs baseline: 3.7695x; 3.7695x over previous
import functools

import jax
import jax.numpy as jnp
from jax import lax
from jax.experimental import pallas as pl
from jax.experimental.pallas import tpu as pltpu
from jax.experimental.pallas import tpu_sc as plsc

N_NODES = 10000
N_FEAT = 128
LANES = 16
N_CORES = 2
N_SUBCORES = 16
N_TILES = N_CORES * N_SUBCORES
CHUNK = 128
ROW_STRIDE = 624
ROWS_PER_TILE = 640


def _matmul_body(x_ref, wl_ref, wr_ref, b_ref, h_ref, dense_ref):
    x = x_ref[...]
    h_ref[...] = jnp.dot(x, wl_ref[...], preferred_element_type=jnp.float32)
    dense_ref[...] = (
        jnp.dot(x, wr_ref[...], preferred_element_type=jnp.float32) + b_ref[...]
    )


def _combine_body(p_ref, d_ref, o_ref):
    o_ref[...] = p_ref[0] + p_ref[1] + d_ref[...]


def _sc_body(chunks_per_tile, h_hbm, src_hbm, dst_hbm, ew_hbm, outp_hbm,
             acc, src_v, dst_v, ew_v, rows_v):
    cid = lax.axis_index("c")
    sid = lax.axis_index("s")
    wid = cid * N_SUBCORES + sid

    def zero_row(e, _):
        for f in range(N_FEAT // LANES):
            rows_v[e, pl.ds(f * LANES, LANES)] = jnp.zeros((LANES,), jnp.float32)
        return 0
    lax.fori_loop(0, CHUNK, zero_row, 0)
    row_base = sid * ROW_STRIDE
    for k in range(ROWS_PER_TILE // CHUNK):
        pltpu.sync_copy(rows_v, acc.at[pl.ds(row_base + k * CHUNK, CHUNK)])
    plsc.subcore_barrier()

    edge_base = wid * chunks_per_tile * CHUNK

    def chunk_body(i, _):
        off = edge_base + i * CHUNK
        pltpu.sync_copy(src_hbm.at[pl.ds(off, CHUNK)], src_v)
        pltpu.sync_copy(dst_hbm.at[pl.ds(off, CHUNK)], dst_v)
        pltpu.sync_copy(ew_hbm.at[pl.ds(off, CHUNK)], ew_v)
        pltpu.sync_copy(h_hbm.at[src_v], rows_v)

        def scale_group(g, _):
            ew16 = ew_v[pl.ds(g * LANES, LANES)]
            for l in range(LANES):
                w = jnp.full((LANES,), ew16[l], jnp.float32)
                e = g * LANES + l
                for f in range(N_FEAT // LANES):
                    sl = pl.ds(f * LANES, LANES)
                    rows_v[e, sl] = rows_v[e, sl] * w
            return 0
        lax.fori_loop(0, CHUNK // LANES, scale_group, 0)
        pltpu.sync_copy(rows_v, acc.at[dst_v], add=True)
        return 0
    lax.fori_loop(0, chunks_per_tile, chunk_body, 0)
    plsc.subcore_barrier()

    pltpu.sync_copy(acc.at[pl.ds(row_base, ROWS_PER_TILE)],
                    outp_hbm.at[cid, pl.ds(row_base, ROWS_PER_TILE)])


def kernel(x, edge_index, edge_weight, W_l, W_r, bias):
    n, f = x.shape
    e = edge_weight.shape[0]
    src = edge_index[0].astype(jnp.int32)
    dst = edge_index[1].astype(jnp.int32)
    ew = edge_weight.astype(jnp.float32)

    chunks_per_tile = -(-e // (N_TILES * CHUNK))
    e_pad = N_TILES * chunks_per_tile * CHUNK
    pad = e_pad - e
    if pad:
        src = jnp.pad(src, (0, pad))
        dst = jnp.pad(dst, (0, pad))
        ew = jnp.pad(ew, (0, pad))

    blk = 2000
    grid = n // blk
    h, dense = pl.pallas_call(
        _matmul_body,
        grid=(grid,),
        in_specs=[
            pl.BlockSpec((blk, f), lambda i: (i, 0)),
            pl.BlockSpec((f, N_FEAT), lambda i: (0, 0)),
            pl.BlockSpec((f, N_FEAT), lambda i: (0, 0)),
            pl.BlockSpec((1, N_FEAT), lambda i: (0, 0)),
        ],
        out_specs=[
            pl.BlockSpec((blk, N_FEAT), lambda i: (i, 0)),
            pl.BlockSpec((blk, N_FEAT), lambda i: (i, 0)),
        ],
        out_shape=[
            jax.ShapeDtypeStruct((n, N_FEAT), jnp.float32),
            jax.ShapeDtypeStruct((n, N_FEAT), jnp.float32),
        ],
    )(x, W_l, W_r, bias.reshape(1, N_FEAT))

    mesh = plsc.VectorSubcoreMesh(core_axis_name="c", subcore_axis_name="s")
    sc_fn = pl.kernel(
        functools.partial(_sc_body, chunks_per_tile),
        out_type=jax.ShapeDtypeStruct((N_CORES, n, N_FEAT), jnp.float32),
        mesh=mesh,
        scratch_types=[
            pltpu.VMEM_SHARED((n, N_FEAT), jnp.float32),
            pltpu.VMEM((CHUNK,), jnp.int32),
            pltpu.VMEM((CHUNK,), jnp.int32),
            pltpu.VMEM((CHUNK,), jnp.float32),
            pltpu.VMEM((CHUNK, N_FEAT), jnp.float32),
        ],
    )
    outp = sc_fn(h, src, dst, ew)

    out = pl.pallas_call(
        _combine_body,
        grid=(grid,),
        in_specs=[
            pl.BlockSpec((N_CORES, blk, N_FEAT), lambda i: (0, i, 0)),
            pl.BlockSpec((blk, N_FEAT), lambda i: (i, 0)),
        ],
        out_specs=pl.BlockSpec((blk, N_FEAT), lambda i: (i, 0)),
        out_shape=jax.ShapeDtypeStruct((n, N_FEAT), jnp.float32),
    )(outp, dense)
    return out


# speedup vs baseline: 6.8217x; 1.8097x over previous
import functools

import jax
import jax.numpy as jnp
from jax import lax
from jax.experimental import pallas as pl
from jax.experimental.pallas import tpu as pltpu
from jax.experimental.pallas import tpu_sc as plsc

N_NODES = 10000
N_FEAT = 128
LANES = 16
N_CORES = 2
N_SUBCORES = 16
N_TILES = N_CORES * N_SUBCORES
CHUNK = 112
NBUF = 3
IBUF = 6
ROW_STRIDE = 624
ROWS_PER_TILE = 640


def _matmul_body(x_ref, wl_ref, wr_ref, b_ref, h_ref, dense_ref):
    x = x_ref[...]
    h_ref[...] = jnp.dot(x, wl_ref[...], preferred_element_type=jnp.float32)
    dense_ref[...] = (
        jnp.dot(x, wr_ref[...], preferred_element_type=jnp.float32) + b_ref[...]
    )


def _combine_body(p_ref, d_ref, o_ref):
    o_ref[...] = p_ref[0] + p_ref[1] + d_ref[...]


def _sc_body(cpt, h_hbm, ipack_hbm, outp_hbm, acc, rows, ips, gsem, ssem, isem):
    cid = lax.axis_index("c")
    sid = lax.axis_index("s")
    wid = cid * N_SUBCORES + sid
    chunk_base = wid * cpt

    def fetch_ipack(j, s):
        pltpu.async_copy(ipack_hbm.at[chunk_base + j], ips[s], isem[s])

    def wait_ipack(j, s):
        pltpu.make_async_copy(ipack_hbm.at[chunk_base + j], ips[s], isem[s]).wait()

    def start_gather(j, s, b):
        pltpu.async_copy(h_hbm.at[ips[s].at[0]], rows[b], gsem[b])

    def wait_gather(j, s, b):
        pltpu.make_async_copy(h_hbm.at[ips[s].at[0]], rows[b], gsem[b]).wait()

    def start_scatter(j, s, b):
        pltpu.async_copy(rows[b], acc.at[ips[s].at[1]], ssem[b], add=True)

    def wait_scatter(j, s, b):
        pltpu.make_async_copy(rows[b], acc.at[ips[s].at[1]], ssem[b]).wait()

    for j in range(4):
        fetch_ipack(j, j)

    def zero_row(e, _):
        for f in range(N_FEAT // LANES):
            rows[0][e, pl.ds(f * LANES, LANES)] = jnp.zeros((LANES,), jnp.float32)
        return 0
    lax.fori_loop(0, CHUNK, zero_row, 0)
    row_base = sid * ROW_STRIDE
    for k in range(ROWS_PER_TILE // CHUNK):
        pltpu.sync_copy(rows[0], acc.at[pl.ds(row_base + k * CHUNK, CHUNK)])
    rem = ROWS_PER_TILE % CHUNK
    if rem:
        nfull = ROWS_PER_TILE // CHUNK
        pltpu.sync_copy(rows[0].at[pl.ds(0, rem)],
                        acc.at[pl.ds(row_base + nfull * CHUNK, rem)])

    for j in range(2):
        wait_ipack(j, j)
        start_gather(j, j, j)
    plsc.subcore_barrier()

    n_groups = cpt // IBUF

    def step(g, _):
        for p in range(IBUF):
            i = g * IBUF + p
            b = p % NBUF
            s = p

            sj = (p + 4) % IBUF
            if p < 2:
                fetch_ipack(i + 4, sj)
            else:
                @pl.when(g < n_groups - 1)
                def _():
                    fetch_ipack(i + 4, sj)

            wait_gather(i, s, b)

            def scale_group(q, _):
                ew16 = lax.bitcast_convert_type(
                    ips[s][2, pl.ds(q * LANES, LANES)], jnp.float32)
                for l in range(LANES):
                    w = jnp.full((LANES,), ew16[l], jnp.float32)
                    for f in range(N_FEAT // LANES):
                        sl = pl.ds(f * LANES, LANES)
                        rows[b][q * LANES + l, sl] = rows[b][q * LANES + l, sl] * w
                return 0
            lax.fori_loop(0, CHUNK // LANES, scale_group, 0)
            start_scatter(i, s, b)

            t = (p + 2) % NBUF
            s2 = (p + 2) % IBUF
            sp = (p + 5) % IBUF
            if p < 4:
                if p == 0:
                    @pl.when(g > 0)
                    def _():
                        wait_scatter(i - 1, sp, t)
                else:
                    wait_scatter(i - 1, sp, t)
                wait_ipack(i + 2, s2)
                start_gather(i + 2, s2, t)
            else:
                @pl.when(g < n_groups - 1)
                def _():
                    wait_scatter(i - 1, sp, t)
                    wait_ipack(i + 2, s2)
                    start_gather(i + 2, s2, t)
        return 0
    lax.fori_loop(0, n_groups, step, 0)

    for b in range(NBUF):
        j = cpt - NBUF + b
        wait_scatter(j, j % IBUF, b)
    plsc.subcore_barrier()

    pltpu.sync_copy(acc.at[pl.ds(row_base, ROWS_PER_TILE)],
                    outp_hbm.at[cid, pl.ds(row_base, ROWS_PER_TILE)])


def kernel(x, edge_index, edge_weight, W_l, W_r, bias):
    n, f = x.shape
    e = edge_weight.shape[0]
    src = edge_index[0].astype(jnp.int32)
    dst = edge_index[1].astype(jnp.int32)
    ew = edge_weight.astype(jnp.float32)

    cpt = -(-e // (N_TILES * CHUNK))
    cpt = -(-cpt // IBUF) * IBUF
    e_pad = N_TILES * cpt * CHUNK
    pad = e_pad - e
    if pad:
        src = jnp.pad(src, (0, pad))
        dst = jnp.pad(dst, (0, pad))
        ew = jnp.pad(ew, (0, pad))
    ipack = jnp.stack(
        [src.reshape(-1, CHUNK), dst.reshape(-1, CHUNK),
         lax.bitcast_convert_type(ew, jnp.int32).reshape(-1, CHUNK)], axis=1)

    blk = 2000
    grid = n // blk
    h, dense = pl.pallas_call(
        _matmul_body,
        grid=(grid,),
        in_specs=[
            pl.BlockSpec((blk, f), lambda i: (i, 0)),
            pl.BlockSpec((f, N_FEAT), lambda i: (0, 0)),
            pl.BlockSpec((f, N_FEAT), lambda i: (0, 0)),
            pl.BlockSpec((1, N_FEAT), lambda i: (0, 0)),
        ],
        out_specs=[
            pl.BlockSpec((blk, N_FEAT), lambda i: (i, 0)),
            pl.BlockSpec((blk, N_FEAT), lambda i: (i, 0)),
        ],
        out_shape=[
            jax.ShapeDtypeStruct((n, N_FEAT), jnp.float32),
            jax.ShapeDtypeStruct((n, N_FEAT), jnp.float32),
        ],
    )(x, W_l, W_r, bias.reshape(1, N_FEAT))

    mesh = plsc.VectorSubcoreMesh(core_axis_name="c", subcore_axis_name="s")

    def sc_entry(h_a, ipack_a, outp_a, acc, r0, r1, r2, i0, i1, i2, i3, i4, i5,
                 g0, g1, g2, s0, s1, s2, q0, q1, q2, q3, q4, q5):
        _sc_body(cpt, h_a, ipack_a, outp_a, acc,
                 (r0, r1, r2), (i0, i1, i2, i3, i4, i5),
                 (g0, g1, g2), (s0, s1, s2), (q0, q1, q2, q3, q4, q5))

    sc_fn = pl.kernel(
        sc_entry,
        out_type=jax.ShapeDtypeStruct((N_CORES, n, N_FEAT), jnp.float32),
        mesh=mesh,
        scratch_types=(
            [pltpu.VMEM_SHARED((n, N_FEAT), jnp.float32)]
            + [pltpu.VMEM((CHUNK, N_FEAT), jnp.float32)] * NBUF
            + [pltpu.VMEM((3, CHUNK), jnp.int32)] * IBUF
            + [pltpu.SemaphoreType.DMA] * (2 * NBUF + IBUF)
        ),
    )
    outp = sc_fn(h, ipack)

    out = pl.pallas_call(
        _combine_body,
        grid=(grid,),
        in_specs=[
            pl.BlockSpec((N_CORES, blk, N_FEAT), lambda i: (0, i, 0)),
            pl.BlockSpec((blk, N_FEAT), lambda i: (i, 0)),
        ],
        out_specs=pl.BlockSpec((blk, N_FEAT), lambda i: (i, 0)),
        out_shape=jax.ShapeDtypeStruct((n, N_FEAT), jnp.float32),
    )(outp, dense)
    return out


# speedup vs baseline: 7.5529x; 1.1072x over previous
import functools

import jax
import jax.numpy as jnp
from jax import lax
from jax.experimental import pallas as pl
from jax.experimental.pallas import tpu as pltpu
from jax.experimental.pallas import tpu_sc as plsc

N_NODES = 10000
N_FEAT = 128
LANES = 16
N_CORES = 2
N_SUBCORES = 16
N_TILES = N_CORES * N_SUBCORES
CHUNK = 112
NBUF = 3
IBUF = 6
ROW_STRIDE = 624
ROWS_PER_TILE = 640


def _matmul_body(x_ref, wl_ref, wr_ref, b_ref, h_ref, dense_ref):
    x = x_ref[...]
    h_ref[...] = jnp.dot(x, wl_ref[...], preferred_element_type=jnp.float32)
    dense_ref[...] = (
        jnp.dot(x, wr_ref[...], preferred_element_type=jnp.float32) + b_ref[...]
    )


def _combine_body(p_ref, d_ref, o_ref):
    o_ref[...] = p_ref[0] + p_ref[1] + d_ref[...]


def _sc_body(cpt0, cpt1, h_hbm, ipack_hbm, outp_hbm, acc, rows, ips,
             gsem, ssem, isem):
    cid = lax.axis_index("c")
    sid = lax.axis_index("s")
    cpt = jnp.where(cid == 0, cpt0, cpt1)
    n_groups = jnp.where(cid == 0, cpt0 // IBUF, cpt1 // IBUF)
    chunk_base = jnp.where(cid == 0, sid * cpt0,
                           N_SUBCORES * cpt0 + sid * cpt1)

    def fetch_ipack(j, s):
        pltpu.async_copy(ipack_hbm.at[chunk_base + j], ips[s], isem[s])

    def wait_ipack(j, s):
        pltpu.make_async_copy(ipack_hbm.at[chunk_base + j], ips[s], isem[s]).wait()

    def start_gather(j, s, b):
        pltpu.async_copy(h_hbm.at[ips[s].at[0]], rows[b], gsem[b])

    def wait_gather(j, s, b):
        pltpu.make_async_copy(h_hbm.at[ips[s].at[0]], rows[b], gsem[b]).wait()

    def start_scatter(j, s, b):
        pltpu.async_copy(rows[b], acc.at[ips[s].at[1]], ssem[b], add=True)

    def wait_scatter(j, s, b):
        pltpu.make_async_copy(rows[b], acc.at[ips[s].at[1]], ssem[b]).wait()

    for j in range(4):
        fetch_ipack(j, j)

    def zero_row(e, _):
        for f in range(N_FEAT // LANES):
            rows[0][e, pl.ds(f * LANES, LANES)] = jnp.zeros((LANES,), jnp.float32)
        return 0
    lax.fori_loop(0, CHUNK, zero_row, 0)
    row_base = sid * ROW_STRIDE
    for k in range(ROWS_PER_TILE // CHUNK):
        pltpu.sync_copy(rows[0], acc.at[pl.ds(row_base + k * CHUNK, CHUNK)])
    rem = ROWS_PER_TILE % CHUNK
    if rem:
        nfull = ROWS_PER_TILE // CHUNK
        pltpu.sync_copy(rows[0].at[pl.ds(0, rem)],
                        acc.at[pl.ds(row_base + nfull * CHUNK, rem)])

    for j in range(2):
        wait_ipack(j, j)
        start_gather(j, j, j)
    plsc.subcore_barrier()

    def step(g, _):
        for p in range(IBUF):
            i = g * IBUF + p
            b = p % NBUF
            s = p

            sj = (p + 4) % IBUF
            if p < 2:
                fetch_ipack(i + 4, sj)
            else:
                @pl.when(g < n_groups - 1)
                def _():
                    fetch_ipack(i + 4, sj)

            wait_gather(i, s, b)

            def scale_group(q, _):
                ew16 = lax.bitcast_convert_type(
                    ips[s][2, pl.ds(q * LANES, LANES)], jnp.float32)
                for l in range(LANES):
                    w = jnp.full((LANES,), ew16[l], jnp.float32)
                    for f in range(N_FEAT // LANES):
                        sl = pl.ds(f * LANES, LANES)
                        rows[b][q * LANES + l, sl] = rows[b][q * LANES + l, sl] * w
                return 0
            lax.fori_loop(0, CHUNK // LANES, scale_group, 0)
            start_scatter(i, s, b)

            t = (p + 2) % NBUF
            s2 = (p + 2) % IBUF
            sp = (p + 5) % IBUF
            if p < 4:
                if p == 0:
                    @pl.when(g > 0)
                    def _():
                        wait_scatter(i - 1, sp, t)
                else:
                    wait_scatter(i - 1, sp, t)
                wait_ipack(i + 2, s2)
                start_gather(i + 2, s2, t)
            else:
                @pl.when(g < n_groups - 1)
                def _():
                    wait_scatter(i - 1, sp, t)
                    wait_ipack(i + 2, s2)
                    start_gather(i + 2, s2, t)
        return 0
    lax.fori_loop(0, n_groups, step, 0)

    for b in range(NBUF):
        wait_scatter(cpt - NBUF + b, (b + NBUF) % IBUF, b)
    plsc.subcore_barrier()

    pltpu.sync_copy(acc.at[pl.ds(row_base, ROWS_PER_TILE)],
                    outp_hbm.at[cid, pl.ds(row_base, ROWS_PER_TILE)])


def kernel(x, edge_index, edge_weight, W_l, W_r, bias):
    n, f = x.shape
    e = edge_weight.shape[0]
    src = edge_index[0].astype(jnp.int32)
    dst = edge_index[1].astype(jnp.int32)
    ew = edge_weight.astype(jnp.float32)

    unit = N_SUBCORES * CHUNK
    cpt1 = -(-e // (3 * unit * IBUF)) * IBUF
    cpt0 = 2 * cpt1
    e_pad = unit * (cpt0 + cpt1)
    pad = e_pad - e
    if pad:
        src = jnp.pad(src, (0, pad))
        dst = jnp.pad(dst, (0, pad))
        ew = jnp.pad(ew, (0, pad))
    ipack = jnp.stack(
        [src.reshape(-1, CHUNK), dst.reshape(-1, CHUNK),
         lax.bitcast_convert_type(ew, jnp.int32).reshape(-1, CHUNK)], axis=1)

    blk = 2000
    grid = n // blk
    h, dense = pl.pallas_call(
        _matmul_body,
        grid=(grid,),
        in_specs=[
            pl.BlockSpec((blk, f), lambda i: (i, 0)),
            pl.BlockSpec((f, N_FEAT), lambda i: (0, 0)),
            pl.BlockSpec((f, N_FEAT), lambda i: (0, 0)),
            pl.BlockSpec((1, N_FEAT), lambda i: (0, 0)),
        ],
        out_specs=[
            pl.BlockSpec((blk, N_FEAT), lambda i: (i, 0)),
            pl.BlockSpec((blk, N_FEAT), lambda i: (i, 0)),
        ],
        out_shape=[
            jax.ShapeDtypeStruct((n, N_FEAT), jnp.float32),
            jax.ShapeDtypeStruct((n, N_FEAT), jnp.float32),
        ],
    )(x, W_l, W_r, bias.reshape(1, N_FEAT))

    mesh = plsc.VectorSubcoreMesh(core_axis_name="c", subcore_axis_name="s")

    def sc_entry(h_a, ipack_a, outp_a, acc, r0, r1, r2, i0, i1, i2, i3, i4, i5,
                 g0, g1, g2, s0, s1, s2, q0, q1, q2, q3, q4, q5):
        _sc_body(cpt0, cpt1, h_a, ipack_a, outp_a, acc,
                 (r0, r1, r2), (i0, i1, i2, i3, i4, i5),
                 (g0, g1, g2), (s0, s1, s2), (q0, q1, q2, q3, q4, q5))

    sc_fn = pl.kernel(
        sc_entry,
        out_type=jax.ShapeDtypeStruct((N_CORES, n, N_FEAT), jnp.float32),
        mesh=mesh,
        scratch_types=(
            [pltpu.VMEM_SHARED((n, N_FEAT), jnp.float32)]
            + [pltpu.VMEM((CHUNK, N_FEAT), jnp.float32)] * NBUF
            + [pltpu.VMEM((3, CHUNK), jnp.int32)] * IBUF
            + [pltpu.SemaphoreType.DMA] * (2 * NBUF + IBUF)
        ),
    )
    outp = sc_fn(h, ipack)

    out = pl.pallas_call(
        _combine_body,
        grid=(grid,),
        in_specs=[
            pl.BlockSpec((N_CORES, blk, N_FEAT), lambda i: (0, i, 0)),
            pl.BlockSpec((blk, N_FEAT), lambda i: (i, 0)),
        ],
        out_specs=pl.BlockSpec((blk, N_FEAT), lambda i: (i, 0)),
        out_shape=jax.ShapeDtypeStruct((n, N_FEAT), jnp.float32),
    )(outp, dense)
    return out
